# baseline (speedup 1.0000x reference)
import functools

import jax
import jax.numpy as jnp
from jax import lax
from jax.experimental import pallas as pl
from jax.experimental.pallas import tpu as pltpu
from jax.experimental.pallas import tpu_sc as plsc

_LANES = 16
_BLK_R = 16


def _sc_add(x2, pe):
    rows, d = x2.shape
    seq_blocks = pe.shape[0] // _BLK_R
    mesh = plsc.VectorSubcoreMesh(core_axis_name="core", subcore_axis_name="subcore")

    @functools.partial(
        pl.kernel,
        out_type=jax.ShapeDtypeStruct((rows, d), x2.dtype),
        mesh=mesh,
    )
    def run(x_hbm, pe_hbm, o_hbm):
        def body(x_vmem, pe_vmem, o_vmem):
            @pl.loop(0, _BLK_R)
            def _row(r):
                @pl.loop(0, d, step=_LANES)
                def _col(c):
                    slc = (pl.ds(r, 1), pl.ds(c, _LANES))
                    o_vmem.at[slc][...] = x_vmem.at[slc][...] + pe_vmem.at[slc][...]

        pltpu.emit_pipeline(
            body,
            grid=(rows // _BLK_R,),
            in_specs=[
                pl.BlockSpec((_BLK_R, d), lambda i: (i, 0)),
                pl.BlockSpec((_BLK_R, d), lambda i: (lax.rem(i, seq_blocks), 0)),
            ],
            out_specs=[pl.BlockSpec((_BLK_R, d), lambda i: (i, 0))],
            core_axis_name=("core", "subcore"),
            dimension_semantics=(pltpu.PARALLEL,),
        )(x_hbm, pe_hbm, o_hbm)

    return run(x2, pe)


def kernel(x, pos_embedding):
    b, s, d = x.shape
    pe = pos_embedding[:s]
    out2 = _sc_add(x.reshape(b * s, d), pe)
    return out2.reshape(b, s, d)


# speedup vs baseline: 1.0962x; 1.0962x over previous
import functools

import jax
import jax.numpy as jnp
from jax import lax
from jax.experimental import pallas as pl
from jax.experimental.pallas import tpu as pltpu
from jax.experimental.pallas import tpu_sc as plsc

_LANES = 16
_BLK_R = 16


def _sc_add(x2, pe, batch):
    rows, d = x2.shape
    seq_blocks = pe.shape[0] // _BLK_R
    mesh = plsc.VectorSubcoreMesh(core_axis_name="core", subcore_axis_name="subcore")

    def x_map(i):
        return (lax.rem(i, batch) * seq_blocks + lax.div(i, batch), 0)

    def pe_map(i):
        return (lax.div(i, batch), 0)

    @functools.partial(
        pl.kernel,
        out_type=jax.ShapeDtypeStruct((rows, d), x2.dtype),
        mesh=mesh,
    )
    def run(x_hbm, pe_hbm, o_hbm):
        def body(x_vmem, pe_vmem, o_vmem):
            @pl.loop(0, _BLK_R)
            def _row(r):
                @pl.loop(0, d, step=_LANES, unroll=8)
                def _col(c):
                    slc = (pl.ds(r, 1), pl.ds(c, _LANES))
                    o_vmem.at[slc][...] = x_vmem.at[slc][...] + pe_vmem.at[slc][...]

        pltpu.emit_pipeline(
            body,
            grid=(rows // _BLK_R,),
            in_specs=[
                pl.BlockSpec((_BLK_R, d), x_map),
                pl.BlockSpec((_BLK_R, d), pe_map),
            ],
            out_specs=[pl.BlockSpec((_BLK_R, d), x_map)],
            core_axis_name=("core", "subcore"),
            dimension_semantics=(pltpu.PARALLEL,),
        )(x_hbm, pe_hbm, o_hbm)

    return run(x2, pe)


def kernel(x, pos_embedding):
    b, s, d = x.shape
    pe = pos_embedding[:s]
    out2 = _sc_add(x.reshape(b * s, d), pe, b)
    return out2.reshape(b, s, d)


# speedup vs baseline: 3.0786x; 2.8085x over previous
import functools

import jax
import jax.numpy as jnp
from jax import lax
from jax.experimental import pallas as pl
from jax.experimental.pallas import tpu as pltpu
from jax.experimental.pallas import tpu_sc as plsc

_LANES = 16
_BLK_R = 16
_TC_BLK_S = 512
_SC_PREFIX = 512


def _sc_add_prefix(x2, pe, batch, prefix):
    rows, d = x2.shape
    seq = pe.shape[0]
    seq_blocks = seq // _BLK_R
    mesh = plsc.VectorSubcoreMesh(core_axis_name="core", subcore_axis_name="subcore")

    def x_map(i):
        return (lax.rem(i, batch) * seq_blocks + lax.div(i, batch), 0)

    def pe_map(i):
        return (lax.div(i, batch), 0)

    def o_map(i):
        return (lax.rem(i, batch), lax.div(i, batch), 0)

    @functools.partial(
        pl.kernel,
        out_type=jax.ShapeDtypeStruct((batch, prefix, d), x2.dtype),
        mesh=mesh,
    )
    def run(x_hbm, pe_hbm, o_hbm):
        def body(x_vmem, pe_vmem, o_vmem):
            @pl.loop(0, _BLK_R)
            def _row(r):
                @pl.loop(0, d, step=_LANES, unroll=8)
                def _col(c):
                    slc = (pl.ds(r, 1), pl.ds(c, _LANES))
                    o_vmem.at[slc][...] = x_vmem.at[slc][...] + pe_vmem.at[slc][...]

        pltpu.emit_pipeline(
            body,
            grid=(batch * prefix // _BLK_R,),
            in_specs=[
                pl.BlockSpec((_BLK_R, d), x_map),
                pl.BlockSpec((_BLK_R, d), pe_map),
            ],
            out_specs=[pl.BlockSpec((None, _BLK_R, d), o_map)],
            core_axis_name=("core", "subcore"),
            dimension_semantics=(pltpu.PARALLEL,),
        )(x_hbm, pe_hbm, o_hbm)

    return run(x2, pe)


def _tc_add_tail(x, pe, seq_off):
    b, s, d = x.shape
    k = seq_off // _TC_BLK_S

    def body(x_ref, pe_ref, o_ref):
        o_ref[...] = x_ref[...] + pe_ref[...]

    xo_map = lambda i, bb: (bb, i + k, 0)
    return pl.pallas_call(
        body,
        grid=(s // _TC_BLK_S - k, b),
        in_specs=[
            pl.BlockSpec((1, _TC_BLK_S, d), xo_map),
            pl.BlockSpec((_TC_BLK_S, d), lambda i, bb: (i + k, 0)),
        ],
        out_specs=pl.BlockSpec((1, _TC_BLK_S, d), xo_map),
        out_shape=jax.ShapeDtypeStruct((b, s, d), x.dtype),
    )(x, pe)


def kernel(x, pos_embedding):
    b, s, d = x.shape
    pe = pos_embedding[:s]
    x2 = x.reshape(b * s, d)
    sc_out = _sc_add_prefix(x2, pe, b, _SC_PREFIX)
    tc_full = _tc_add_tail(x, pe, _SC_PREFIX)
    return lax.dynamic_update_slice(tc_full, sc_out, (0, 0, 0))
